# baseline (speedup 1.0000x reference)
import functools
import jax
import jax.numpy as jnp
from jax import lax
from jax.experimental import pallas as pl
from jax.experimental.pallas import tpu as pltpu, tpu_sc as plsc

BATCH = 1024
L = 200
N = 16
D = 8
EMB = N * D
T = BATCH * L

NC = 2
NS = 16
NW = NC * NS
TPW = T // NW

C = 640
K = C // 128
NCHUNK = TPW // C


def _sc_expand_kernel(w_hbm, x_hbm, perm_hbm, scale_hbm, out_hbm,
                      idx_v, rows_v, out_v, perm_v, scale_v, sem):
    wid = lax.axis_index("s") * NC + lax.axis_index("c")

    pltpu.sync_copy(perm_hbm, perm_v)
    pltpu.sync_copy(scale_hbm, scale_v)
    perm_regs = [perm_v[pl.ds(16 * g, 16)] for g in range(D)]
    scale_regs = [scale_v[pl.ds(16 * g, 16)] for g in range(D)]

    tokw0 = wid * TPW

    def chunk_body(ci, carry):
        tok0 = tokw0 + ci * C
        pltpu.sync_copy(x_hbm.at[pl.ds(tok0, C)], idx_v)
        copies = [
            pltpu.async_copy(w_hbm.at[idx_v.at[pl.ds(j * 128, 128)]],
                             rows_v.at[pl.ds(j * 128, 128)], sem)
            for j in range(K)
        ]
        for c in copies:
            c.wait()

        dnums = lax.GatherDimensionNumbers(
            offset_dims=(), collapsed_slice_dims=(0,), start_index_map=(0,))

        def tok_body(t, tc):
            emb = rows_v[t]
            for g in range(D):
                vals = lax.gather(
                    emb, perm_regs[g][:, None], dnums, slice_sizes=(1,),
                    mode=lax.GatherScatterMode.PROMISE_IN_BOUNDS)
                out_v[t, pl.ds(16 * g, 16)] = vals * scale_regs[g]
            return tc

        lax.fori_loop(0, C, tok_body, 0, unroll=2)
        pltpu.sync_copy(out_v, out_hbm.at[pl.ds(tok0, C)])
        return carry

    lax.fori_loop(0, NCHUNK, chunk_body, 0)


@jax.jit
def _run(w, x1, perm_idx, scale):
    mesh = plsc.VectorSubcoreMesh(core_axis_name="c", subcore_axis_name="s")
    kfn = functools.partial(
        pl.kernel,
        out_type=jax.ShapeDtypeStruct((T, EMB), jnp.float32),
        mesh=mesh,
        scratch_types=[
            pltpu.VMEM((C,), jnp.int32),
            pltpu.VMEM((C, N), jnp.float32),
            pltpu.VMEM((C, EMB), jnp.float32),
            pltpu.VMEM((EMB,), jnp.int32),
            pltpu.VMEM((EMB,), jnp.float32),
            pltpu.SemaphoreType.DMA,
        ],
        compiler_params=pltpu.CompilerParams(use_tc_tiling_on_sc=False),
    )(_sc_expand_kernel)
    return kfn(w, x1, perm_idx, scale)


def kernel(x, W, B, p):
    p = p.astype(jnp.int32)
    perm_idx = p // D
    scale = B[0, p % D].astype(jnp.float32)
    x1 = x.astype(jnp.int32).reshape(T)
    out = _run(W, x1, perm_idx, scale)
    return out.reshape(BATCH, L, EMB)
